# baseline (speedup 1.0000x reference)
import jax
import jax.numpy as jnp
from jax.experimental import pallas as pl
from jax.experimental.pallas import tpu as pltpu

_IN_F = 4
_OUT_F = 2
_PACK = 32
_HGRP = 64
_TBR = 1024


def _round_up(n, m):
    return (n + m - 1) // m * m


def _packed_mlp_kernel(x_ref, t2_ref, b1_ref, w2_ref, b2_ref, o_ref):
    h = jnp.dot(x_ref[...], t2_ref[...],
                preferred_element_type=jnp.float32)
    h = jnp.maximum(h + b1_ref[...], 0.0)
    y = jnp.dot(h, w2_ref[...],
                preferred_element_type=jnp.float32)
    o_ref[...] = y + b2_ref[...]


def kernel(x, w1p, b1p, w2p, b2p):
    w1t = w1p[:_HGRP].T
    w2t = w2p[:, :_HGRP].T
    eye = jnp.eye(_PACK, dtype=jnp.float32)
    t2 = jnp.kron(eye, w1t)
    w2b = jnp.kron(eye, w2t)
    b1b = jnp.tile(b1p[:_HGRP, 0], _PACK)[None, :]
    b2b = jnp.tile(b2p[:, 0], _PACK)[None, :]

    B = x.shape[0]
    rows = _PACK * _TBR
    b_pad = _round_up(B, rows)
    if b_pad != B:
        x = jnp.pad(x, ((0, b_pad - B), (0, 0)))
    xp = x.reshape(b_pad // _PACK, _PACK * _IN_F)

    out = pl.pallas_call(
        _packed_mlp_kernel,
        out_shape=jax.ShapeDtypeStruct(
            (b_pad // _PACK, _PACK * _OUT_F), jnp.float32),
        grid=(b_pad // rows,),
        in_specs=[
            pl.BlockSpec((_TBR, _PACK * _IN_F), lambda i: (i, 0)),
            pl.BlockSpec(t2.shape, lambda i: (0, 0)),
            pl.BlockSpec(b1b.shape, lambda i: (0, 0)),
            pl.BlockSpec(w2b.shape, lambda i: (0, 0)),
            pl.BlockSpec(b2b.shape, lambda i: (0, 0)),
        ],
        out_specs=pl.BlockSpec((_TBR, _PACK * _OUT_F), lambda i: (i, 0)),
        compiler_params=pltpu.CompilerParams(
            dimension_semantics=("parallel",)),
    )(xp, t2, b1b, w2b, b2b)

    return out.reshape(b_pad, _OUT_F)[:B]


# speedup vs baseline: 20.3279x; 20.3279x over previous
import jax
import jax.numpy as jnp
from jax.experimental import pallas as pl
from jax.experimental.pallas import tpu as pltpu

_HID = 64
_TBL = 65536
_TBC = 2048


def _round_up(n, m):
    return (n + m - 1) // m * m


def _mlp_lanes_kernel(xt_ref, w1_ref, b1_ref, w2_ref, b2_ref, ot_ref):
    w1 = w1_ref[...]
    b1 = b1_ref[...]
    w2 = w2_ref[...]
    b2 = b2_ref[...]
    tbl = xt_ref.shape[1]
    for c in range(0, tbl, _TBC):
        w = min(_TBC, tbl - c)
        xc = xt_ref[:, c:c + w]
        h = jnp.dot(w1, xc, preferred_element_type=jnp.float32)
        h = jnp.maximum(h + b1, 0.0)
        y = jnp.dot(w2, h, preferred_element_type=jnp.float32)
        ot_ref[:, c:c + w] = y + b2


def kernel(x, w1p, b1p, w2p, b2p):
    w1c = w1p[:_HID]
    b1c = b1p[:_HID]
    w2c = w2p[:, :_HID]

    B = x.shape[0]
    xt = x.T
    b_pad = _round_up(B, 128)
    if b_pad != B:
        xt = jnp.pad(xt, ((0, 0), (0, b_pad - B)))
    if b_pad % _TBL == 0:
        tbl = _TBL
    else:
        tbl = b_pad

    yt = pl.pallas_call(
        _mlp_lanes_kernel,
        out_shape=jax.ShapeDtypeStruct((2, b_pad), jnp.float32),
        grid=(b_pad // tbl,),
        in_specs=[
            pl.BlockSpec((4, tbl), lambda i: (0, i)),
            pl.BlockSpec(w1c.shape, lambda i: (0, 0)),
            pl.BlockSpec(b1c.shape, lambda i: (0, 0)),
            pl.BlockSpec(w2c.shape, lambda i: (0, 0)),
            pl.BlockSpec(b2p.shape, lambda i: (0, 0)),
        ],
        out_specs=pl.BlockSpec((2, tbl), lambda i: (0, i)),
        compiler_params=pltpu.CompilerParams(
            dimension_semantics=("parallel",)),
    )(xt, w1c, b1c, w2c, b2p)

    if b_pad != B:
        yt = yt[:, :B]
    return yt.T


# speedup vs baseline: 21.0982x; 1.0379x over previous
import jax
import jax.numpy as jnp
from jax.experimental import pallas as pl
from jax.experimental.pallas import tpu as pltpu

_HID = 64
_TBL = 131072
_TBC = 2048


def _round_up(n, m):
    return (n + m - 1) // m * m


def _mlp_lanes_kernel(xt_ref, w1_ref, b1_ref, w2_ref, b2_ref, ot_ref):
    w1 = w1_ref[...].astype(jnp.bfloat16)
    b1 = b1_ref[...]
    w2 = w2_ref[...].astype(jnp.bfloat16)
    b2 = b2_ref[...]
    tbl = xt_ref.shape[1]
    for c in range(0, tbl, _TBC):
        w = min(_TBC, tbl - c)
        xc = xt_ref[:, c:c + w].astype(jnp.bfloat16)
        h = jnp.dot(w1, xc, preferred_element_type=jnp.float32)
        h = jnp.maximum(h + b1, 0.0)
        y = jnp.dot(w2, h.astype(jnp.bfloat16),
                    preferred_element_type=jnp.float32)
        ot_ref[:, c:c + w] = y + b2


def kernel(x, w1p, b1p, w2p, b2p):
    w1c = w1p[:_HID]
    b1c = b1p[:_HID]
    w2c = w2p[:, :_HID]

    B = x.shape[0]
    xt = x.T
    b_pad = _round_up(B, 128)
    if b_pad != B:
        xt = jnp.pad(xt, ((0, 0), (0, b_pad - B)))
    if b_pad % _TBL == 0:
        tbl = _TBL
    else:
        tbl = b_pad

    yt = pl.pallas_call(
        _mlp_lanes_kernel,
        out_shape=jax.ShapeDtypeStruct((2, b_pad), jnp.float32),
        grid=(b_pad // tbl,),
        in_specs=[
            pl.BlockSpec((4, tbl), lambda i: (0, i)),
            pl.BlockSpec(w1c.shape, lambda i: (0, 0)),
            pl.BlockSpec(b1c.shape, lambda i: (0, 0)),
            pl.BlockSpec(w2c.shape, lambda i: (0, 0)),
            pl.BlockSpec(b2p.shape, lambda i: (0, 0)),
        ],
        out_specs=pl.BlockSpec((2, tbl), lambda i: (0, i)),
        compiler_params=pltpu.CompilerParams(
            dimension_semantics=("parallel",)),
    )(xt, w1c, b1c, w2c, b2p)

    if b_pad != B:
        yt = yt[:, :B]
    return yt.T


# speedup vs baseline: 38.0549x; 1.8037x over previous
import jax
import jax.numpy as jnp
from jax.experimental import pallas as pl
from jax.experimental.pallas import tpu as pltpu

_HID = 56
_TBL = 131072
_TBC = 8192


def _round_up(n, m):
    return (n + m - 1) // m * m


def _mlp_lanes_kernel(xt_ref, w1_ref, b1_ref, w2_ref, b2_ref, ot_ref):
    w1 = w1_ref[...].astype(jnp.bfloat16)
    b1 = b1_ref[...].astype(jnp.bfloat16)
    w2 = w2_ref[...].astype(jnp.bfloat16)
    b2 = b2_ref[...]
    tbl = xt_ref.shape[1]
    for c in range(0, tbl, _TBC):
        w = min(_TBC, tbl - c)
        xc = xt_ref[:, c:c + w].astype(jnp.bfloat16)
        h = jnp.dot(w1, xc, preferred_element_type=jnp.float32)
        hb = jnp.maximum(h.astype(jnp.bfloat16) + b1, 0)
        y = jnp.dot(w2, hb, preferred_element_type=jnp.float32)
        ot_ref[:, c:c + w] = y + b2


def kernel(x, w1p, b1p, w2p, b2p):
    w1c = w1p[:_HID]
    b1c = b1p[:_HID]
    w2c = w2p[:, :_HID]

    B = x.shape[0]
    xt = x.T
    b_pad = _round_up(B, 128)
    if b_pad != B:
        xt = jnp.pad(xt, ((0, 0), (0, b_pad - B)))
    if b_pad % _TBL == 0:
        tbl = _TBL
    else:
        tbl = b_pad

    yt = pl.pallas_call(
        _mlp_lanes_kernel,
        out_shape=jax.ShapeDtypeStruct((2, b_pad), jnp.float32),
        grid=(b_pad // tbl,),
        in_specs=[
            pl.BlockSpec((4, tbl), lambda i: (0, i)),
            pl.BlockSpec(w1c.shape, lambda i: (0, 0)),
            pl.BlockSpec(b1c.shape, lambda i: (0, 0)),
            pl.BlockSpec(w2c.shape, lambda i: (0, 0)),
            pl.BlockSpec(b2p.shape, lambda i: (0, 0)),
        ],
        out_specs=pl.BlockSpec((2, tbl), lambda i: (0, i)),
        compiler_params=pltpu.CompilerParams(
            dimension_semantics=("parallel",)),
    )(xt, w1c, b1c, w2c, b2p)

    if b_pad != B:
        yt = yt[:, :B]
    return yt.T


# speedup vs baseline: 38.2761x; 1.0058x over previous
import jax
import jax.numpy as jnp
from jax.experimental import pallas as pl
from jax.experimental.pallas import tpu as pltpu

_HID = 56
_TBL = 262144
_TBC = 8192


def _round_up(n, m):
    return (n + m - 1) // m * m


def _mlp_lanes_kernel(xt_ref, w1_ref, b1_ref, w2_ref, b2_ref, ot_ref):
    w1 = w1_ref[...].astype(jnp.bfloat16)
    b1 = b1_ref[...].astype(jnp.bfloat16)
    w2 = w2_ref[...].astype(jnp.bfloat16)
    b2 = b2_ref[...]
    tbl = xt_ref.shape[1]
    for c in range(0, tbl, _TBC):
        w = min(_TBC, tbl - c)
        xc = xt_ref[:, c:c + w].astype(jnp.bfloat16)
        h = jnp.dot(w1, xc, preferred_element_type=jnp.float32)
        hb = jnp.maximum(h.astype(jnp.bfloat16) + b1, 0)
        y = jnp.dot(w2, hb, preferred_element_type=jnp.float32)
        ot_ref[:, c:c + w] = y + b2


def kernel(x, w1p, b1p, w2p, b2p):
    w1c = w1p[:_HID]
    b1c = b1p[:_HID]
    w2c = w2p[:, :_HID]

    B = x.shape[0]
    xt = x.T
    b_pad = _round_up(B, 128)
    if b_pad != B:
        xt = jnp.pad(xt, ((0, 0), (0, b_pad - B)))
    if b_pad % _TBL == 0:
        tbl = _TBL
    else:
        tbl = b_pad

    yt = pl.pallas_call(
        _mlp_lanes_kernel,
        out_shape=jax.ShapeDtypeStruct((2, b_pad), jnp.float32),
        grid=(b_pad // tbl,),
        in_specs=[
            pl.BlockSpec((4, tbl), lambda i: (0, i)),
            pl.BlockSpec(w1c.shape, lambda i: (0, 0)),
            pl.BlockSpec(b1c.shape, lambda i: (0, 0)),
            pl.BlockSpec(w2c.shape, lambda i: (0, 0)),
            pl.BlockSpec(b2p.shape, lambda i: (0, 0)),
        ],
        out_specs=pl.BlockSpec((2, tbl), lambda i: (0, i)),
        compiler_params=pltpu.CompilerParams(
            dimension_semantics=("parallel",)),
    )(xt, w1c, b1c, w2c, b2p)

    if b_pad != B:
        yt = yt[:, :B]
    return yt.T


# speedup vs baseline: 41.3469x; 1.0802x over previous
import jax
import jax.numpy as jnp
from jax.experimental import pallas as pl
from jax.experimental.pallas import tpu as pltpu

_HID = 56
_TBL = 262144
_TBC = 2048
_QUAD = 4 * _TBC


def _round_up(n, m):
    return (n + m - 1) // m * m


def _mlp_lanes_kernel(xt_ref, w1q_ref, b1q_ref, w2q_ref, b2_ref, ot_ref):
    w1q = w1q_ref[...].astype(jnp.bfloat16)
    b1q = b1q_ref[...].astype(jnp.bfloat16)
    w2q = w2q_ref[...].astype(jnp.bfloat16)
    b2 = b2_ref[...]
    tbl = xt_ref.shape[1]
    for q in range(0, tbl, _QUAD):
        w = min(_TBC, tbl - q)
        los = [min(q + a * _TBC, tbl - w) for a in range(4)]
        xq = jnp.concatenate([xt_ref[:, lo:lo + w] for lo in los],
                             axis=0).astype(jnp.bfloat16)
        h = jnp.dot(w1q, xq, preferred_element_type=jnp.float32)
        hb = jnp.maximum(h.astype(jnp.bfloat16) + b1q, 0)
        y4 = jnp.dot(w2q, hb, preferred_element_type=jnp.float32)
        for a, lo in enumerate(los):
            ot_ref[:, lo:lo + w] = y4[2 * a:2 * a + 2, :] + b2


def kernel(x, w1p, b1p, w2p, b2p):
    w1c = w1p[:_HID]
    b1c = b1p[:_HID]
    w2c = w2p[:, :_HID]
    eye4 = jnp.eye(4, dtype=jnp.float32)
    w1q = jnp.kron(eye4, w1c)
    b1q = jnp.tile(b1c, (4, 1))
    w2q = jnp.kron(eye4, w2c)

    B = x.shape[0]
    xt = x.T
    b_pad = _round_up(B, 512)
    if b_pad != B:
        xt = jnp.pad(xt, ((0, 0), (0, b_pad - B)))
    if b_pad % _TBL == 0:
        tbl = _TBL
    else:
        tbl = b_pad

    yt = pl.pallas_call(
        _mlp_lanes_kernel,
        out_shape=jax.ShapeDtypeStruct((2, b_pad), jnp.float32),
        grid=(b_pad // tbl,),
        in_specs=[
            pl.BlockSpec((4, tbl), lambda i: (0, i)),
            pl.BlockSpec(w1q.shape, lambda i: (0, 0)),
            pl.BlockSpec(b1q.shape, lambda i: (0, 0)),
            pl.BlockSpec(w2q.shape, lambda i: (0, 0)),
            pl.BlockSpec(b2p.shape, lambda i: (0, 0)),
        ],
        out_specs=pl.BlockSpec((2, tbl), lambda i: (0, i)),
        compiler_params=pltpu.CompilerParams(
            dimension_semantics=("parallel",)),
    )(xt, w1q, b1q, w2q, b2p)

    if b_pad != B:
        yt = yt[:, :B]
    return yt.T


# speedup vs baseline: 43.6977x; 1.0569x over previous
import jax
import jax.numpy as jnp
from jax.experimental import pallas as pl
from jax.experimental.pallas import tpu as pltpu

_HID = 56
_TBL = 262144
_TBC = 4096
_QUAD = 4 * _TBC


def _round_up(n, m):
    return (n + m - 1) // m * m


def _mlp_lanes_kernel(xt_ref, w1q_ref, b1q_ref, w2q_ref, b2_ref, ot_ref):
    w1q = w1q_ref[...].astype(jnp.bfloat16)
    b1q = b1q_ref[...].astype(jnp.bfloat16)
    w2q = w2q_ref[...].astype(jnp.bfloat16)
    b2 = b2_ref[...]
    tbl = xt_ref.shape[1]
    for q in range(0, tbl, _QUAD):
        w = min(_TBC, tbl - q)
        los = [min(q + a * _TBC, tbl - w) for a in range(4)]
        xq = jnp.concatenate([xt_ref[:, lo:lo + w] for lo in los],
                             axis=0).astype(jnp.bfloat16)
        h = jnp.dot(w1q, xq, preferred_element_type=jnp.float32)
        hb = jnp.maximum(h.astype(jnp.bfloat16) + b1q, 0)
        y4 = jnp.dot(w2q, hb, preferred_element_type=jnp.float32)
        for a, lo in enumerate(los):
            ot_ref[:, lo:lo + w] = y4[2 * a:2 * a + 2, :] + b2


def kernel(x, w1p, b1p, w2p, b2p):
    w1c = w1p[:_HID]
    b1c = b1p[:_HID]
    w2c = w2p[:, :_HID]
    eye4 = jnp.eye(4, dtype=jnp.float32)
    w1q = jnp.kron(eye4, w1c)
    b1q = jnp.tile(b1c, (4, 1))
    w2q = jnp.kron(eye4, w2c)

    B = x.shape[0]
    xt = x.T
    b_pad = _round_up(B, 512)
    if b_pad != B:
        xt = jnp.pad(xt, ((0, 0), (0, b_pad - B)))
    if b_pad % _TBL == 0:
        tbl = _TBL
    else:
        tbl = b_pad

    yt = pl.pallas_call(
        _mlp_lanes_kernel,
        out_shape=jax.ShapeDtypeStruct((2, b_pad), jnp.float32),
        grid=(b_pad // tbl,),
        in_specs=[
            pl.BlockSpec((4, tbl), lambda i: (0, i)),
            pl.BlockSpec(w1q.shape, lambda i: (0, 0)),
            pl.BlockSpec(b1q.shape, lambda i: (0, 0)),
            pl.BlockSpec(w2q.shape, lambda i: (0, 0)),
            pl.BlockSpec(b2p.shape, lambda i: (0, 0)),
        ],
        out_specs=pl.BlockSpec((2, tbl), lambda i: (0, i)),
        compiler_params=pltpu.CompilerParams(
            dimension_semantics=("parallel",)),
    )(xt, w1q, b1q, w2q, b2p)

    if b_pad != B:
        yt = yt[:, :B]
    return yt.T
